# Initial kernel scaffold; baseline (speedup 1.0000x reference)
import jax
import jax.numpy as jnp
from jax.experimental import pallas as pl


def kernel(idx, W, A_prior, freeze_mask):
    raise NotImplementedError("write your pallas kernel here")


# speedup vs baseline: 23.6431x; 23.6431x over previous
import functools

import jax
import jax.numpy as jnp
from jax.experimental import pallas as pl

_N = 8192
_K = 64
_BLOCK_ROWS = 128


def _topk_mask_body(w_ref, a_ref, m_ref, o_ref):
    m = m_ref[...]
    adj = a_ref[...] * m + jnp.maximum(w_ref[...], 0.0) * (1.0 - m)
    bits = jax.lax.bitcast_convert_type(adj, jnp.int32)

    hi = jnp.max(bits, axis=1, keepdims=True) + 1
    lo = jnp.zeros_like(hi)

    def cond(state):
        lo_, hi_ = state
        return jnp.any(hi_ - lo_ > 1)

    def body(state):
        lo_, hi_ = state
        mid = lo_ + ((hi_ - lo_) >> 1)
        cnt = jnp.sum((bits >= mid).astype(jnp.int32), axis=1, keepdims=True)
        ge = cnt >= _K
        return jnp.where(ge, mid, lo_), jnp.where(ge, hi_, mid)

    lo, hi = jax.lax.while_loop(cond, body, (lo, hi))
    o_ref[...] = jnp.where(bits >= lo, adj, 0.0)


@jax.jit
def kernel(idx, W, A_prior, freeze_mask):
    del idx
    grid = (_N // _BLOCK_ROWS,)
    spec = pl.BlockSpec((_BLOCK_ROWS, _N), lambda i: (i, 0))
    return pl.pallas_call(
        _topk_mask_body,
        grid=grid,
        in_specs=[spec, spec, spec],
        out_specs=spec,
        out_shape=jax.ShapeDtypeStruct((_N, _N), jnp.float32),
    )(W, A_prior, freeze_mask)


# speedup vs baseline: 28.1790x; 1.1919x over previous
import functools

import jax
import jax.numpy as jnp
from jax.experimental import pallas as pl

_N = 8192
_K = 64
_BLOCK_ROWS = 128


def _topk_mask_body(w_ref, a_ref, m_ref, o_ref):
    m = m_ref[...]
    relu_w = jnp.maximum(w_ref[...], 0.0)
    adj = relu_w + m * (a_ref[...] - relu_w)
    bits = jax.lax.bitcast_convert_type(adj, jnp.int32)

    gm = bits[:, 0:64]
    for k in range(1, 128):
        gm = jnp.maximum(gm, bits[:, k * 64:(k + 1) * 64])
    lo = jnp.min(gm, axis=1, keepdims=True)
    hi = jnp.max(gm, axis=1, keepdims=True) + 1

    def cond(state):
        lo_, hi_ = state
        return jnp.any(hi_ - lo_ > 1)

    def body(state):
        lo_, hi_ = state
        mid = lo_ + ((hi_ - lo_) >> 1)
        cnt = jnp.sum((bits >= mid).astype(jnp.int32), axis=1, keepdims=True)
        ge = cnt >= _K
        return jnp.where(ge, mid, lo_), jnp.where(ge, hi_, mid)

    lo, hi = jax.lax.while_loop(cond, body, (lo, hi))
    o_ref[...] = jnp.where(bits >= lo, adj, 0.0)


@jax.jit
def kernel(idx, W, A_prior, freeze_mask):
    del idx
    grid = (_N // _BLOCK_ROWS,)
    spec = pl.BlockSpec((_BLOCK_ROWS, _N), lambda i: (i, 0))
    return pl.pallas_call(
        _topk_mask_body,
        grid=grid,
        in_specs=[spec, spec, spec],
        out_specs=spec,
        out_shape=jax.ShapeDtypeStruct((_N, _N), jnp.float32),
    )(W, A_prior, freeze_mask)
